# baseline (speedup 1.0000x reference)
import functools

import jax
import jax.numpy as jnp
from jax import lax
from jax.experimental import pallas as pl
from jax.experimental.pallas import tpu as pltpu
from jax.experimental.pallas import tpu_sc as plsc

HID = 512


def _gather_half(table, i_arr, j_arr, row0, SH, S, T):
    info = plsc.get_sparse_core_info()
    NC, NS = info.num_cores, info.num_subcores
    NW = NC * NS
    per_w = SH // NW
    C = 64
    n_ch = per_w // C
    tsh = (T - 1).bit_length()
    assert T == (1 << tsh)
    mesh = plsc.VectorSubcoreMesh(core_axis_name="c", subcore_axis_name="s")

    @functools.partial(
        pl.kernel,
        mesh=mesh,
        out_type=jax.ShapeDtypeStruct((SH, 4 * HID), jnp.float32),
        scratch_types=[
            pltpu.VMEM((per_w,), jnp.int32),
            pltpu.VMEM((per_w,), jnp.int32),
            pltpu.VMEM((4 * n_ch, C), jnp.int32),
            pltpu.VMEM((C, HID), jnp.float32),
            pltpu.VMEM((C, HID), jnp.float32),
            pltpu.SemaphoreType.DMA,
            pltpu.SemaphoreType.DMA,
        ],
    )
    def k(table_h, i_h, j_h, a_h, iv, jv, idx, buf0, buf1, sem0, sem1):
        wid = lax.axis_index("s") * NC + lax.axis_index("c")
        base = wid * per_w
        pltpu.sync_copy(i_h.at[pl.ds(base, per_w)], iv)
        pltpu.sync_copy(j_h.at[pl.ds(base, per_w)], jv)

        for h in range(n_ch):
            def comp(p, _, h=h):
                off = h * C + p * 16
                i16 = iv[pl.ds(off, 16)]
                j16 = jv[pl.ds(off, 16)]
                lanes = row0 + base + off + lax.iota(jnp.int32, 16)
                q = (lanes >> tsh) << tsh
                e0 = 2 * (q + j16)
                e1 = 2 * (q + i16) + 1
                e2 = 2 * (q + jnp.maximum(i16 - 1, 0))
                e3 = 2 * (q + jnp.minimum(j16 + 1, T - 1)) + 1
                idx[0 * n_ch + h, pl.ds(p * 16, 16)] = e0
                idx[1 * n_ch + h, pl.ds(p * 16, 16)] = e1
                idx[2 * n_ch + h, pl.ds(p * 16, 16)] = e2
                idx[3 * n_ch + h, pl.ds(p * 16, 16)] = e3
                return _

            lax.fori_loop(0, C // 16, comp, None)

        order = [(h, blk) for h in range(n_ch) for blk in range(4)]
        bufs = (buf0, buf1)
        sems = (sem0, sem1)
        prev = None
        for t, (h, blk) in enumerate(order):
            p = t % 2
            cp = pltpu.async_copy(
                table_h.at[idx.at[blk * n_ch + h]], bufs[p], sems[p])
            if prev is not None:
                pt, pcp = prev
                pcp.wait()
                ph, pblk = order[pt]
                pltpu.sync_copy(
                    bufs[pt % 2],
                    a_h.at[pl.ds(base + ph * C, C),
                           pl.ds(pblk * HID, HID)])
            prev = (t, cp)
        pt, pcp = prev
        pcp.wait()
        ph, pblk = order[pt]
        pltpu.sync_copy(
            bufs[pt % 2],
            a_h.at[pl.ds(base + ph * C, C), pl.ds(pblk * HID, HID)])

    return k(table, i_arr, j_arr)


def _matmul_half(A, masks, Wr, b2, S, OUT, m_off, prev_out=None):
    BM = 1024
    K = 4 * HID
    SH = A.shape[0]

    def mm(*refs):
        if prev_out is None:
            a_ref, m_ref, w_ref, b_ref, o_ref = refs
        else:
            a_ref, m_ref, w_ref, b_ref, _prev, o_ref = refs
        parts = []
        for c in range(4):
            blk = a_ref[:, c * HID:(c + 1) * HID] * m_ref[:, c][:, None]
            parts.append(blk.astype(jnp.bfloat16))
        a = jnp.concatenate(parts, axis=1)
        o_ref[...] = lax.dot_general(
            a, w_ref[...],
            (((1,), (1,)), ((), ())),
            preferred_element_type=jnp.float32) + b_ref[...]

    in_specs = [
        pl.BlockSpec((BM, K), lambda m: (m, 0)),
        pl.BlockSpec((BM, 4), lambda m: (m, 0)),
        pl.BlockSpec((OUT, K), lambda m: (0, 0)),
        pl.BlockSpec((1, OUT), lambda m: (0, 0)),
    ]
    operands = [A, masks, Wr, b2]
    kwargs = {}
    if prev_out is not None:
        in_specs.append(pl.BlockSpec(memory_space=pltpu.MemorySpace.HBM))
        operands.append(prev_out)
        kwargs["input_output_aliases"] = {4: 0}

    moff_tiles = m_off // BM
    return pl.pallas_call(
        mm,
        grid=(SH // BM,),
        in_specs=in_specs,
        out_specs=pl.BlockSpec((BM, OUT), lambda m: (m + moff_tiles, 0)),
        out_shape=jax.ShapeDtypeStruct((S, OUT), jnp.float32),
        compiler_params=pltpu.CompilerParams(
            dimension_semantics=("parallel",)),
        **kwargs,
    )(*operands)


def kernel(input, span_idxs, W, b):
    B, T, two_h = input.shape
    OUT = W.shape[0]
    S = B * T
    SH = S // 2

    table = input.reshape(2 * S, HID)

    si = span_idxs.astype(jnp.int32)
    i_arr = si[..., 0].reshape(S)
    j_arr = si[..., 1].reshape(S)

    skip = (i_arr == 0) & (j_arr == 0)
    m01 = ~skip
    m2 = i_arr > 0
    m3 = m01 & (j_arr < T - 1)
    masks = jnp.stack([m01, m01, m2, m3], axis=1).astype(jnp.float32)

    W1 = W[:, 0:HID]
    W2 = W[:, HID:2 * HID]
    W3 = W[:, 2 * HID:3 * HID]
    W4 = W[:, 3 * HID:4 * HID]
    Wr = jnp.concatenate([W1, W2, W3 - W1, W4 - W2],
                         axis=1).astype(jnp.bfloat16)
    b2 = b.reshape(1, OUT)

    NSPLIT = 4
    SH = S // NSPLIT
    parts = [
        _gather_half(table, i_arr[p * SH:(p + 1) * SH],
                     j_arr[p * SH:(p + 1) * SH], p * SH, SH, S, T)
        for p in range(NSPLIT)
    ]
    out = None
    for p in range(NSPLIT):
        out = _matmul_half(parts[p], masks[p * SH:(p + 1) * SH], Wr, b2,
                           S, OUT, p * SH, prev_out=out)
    return out.reshape(B, T, OUT)


# speedup vs baseline: 1.0777x; 1.0777x over previous
import functools

import jax
import jax.numpy as jnp
from jax import lax
from jax.experimental import pallas as pl
from jax.experimental.pallas import tpu as pltpu
from jax.experimental.pallas import tpu_sc as plsc

HID = 512


def _gather_half(table, i_arr, j_arr, row0, SH, S, T):
    info = plsc.get_sparse_core_info()
    NC, NS = info.num_cores, info.num_subcores
    NW = NC * NS
    per_w = SH // NW
    C = 64
    n_ch = per_w // C
    tsh = (T - 1).bit_length()
    assert T == (1 << tsh)
    mesh = plsc.VectorSubcoreMesh(core_axis_name="c", subcore_axis_name="s")

    @functools.partial(
        pl.kernel,
        mesh=mesh,
        out_type=jax.ShapeDtypeStruct((SH, 4 * HID), jnp.float32),
        scratch_types=[
            pltpu.VMEM((per_w,), jnp.int32),
            pltpu.VMEM((per_w,), jnp.int32),
            pltpu.VMEM((4 * n_ch, C), jnp.int32),
            pltpu.VMEM((C, HID), jnp.float32),
            pltpu.VMEM((C, HID), jnp.float32),
            pltpu.SemaphoreType.DMA,
            pltpu.SemaphoreType.DMA,
        ],
    )
    def k(table_h, i_h, j_h, a_h, iv, jv, idx, buf0, buf1, sem0, sem1):
        wid = lax.axis_index("s") * NC + lax.axis_index("c")
        base = wid * per_w
        pltpu.sync_copy(i_h.at[pl.ds(base, per_w)], iv)
        pltpu.sync_copy(j_h.at[pl.ds(base, per_w)], jv)

        for h in range(n_ch):
            def comp(p, _, h=h):
                off = h * C + p * 16
                i16 = iv[pl.ds(off, 16)]
                j16 = jv[pl.ds(off, 16)]
                lanes = row0 + base + off + lax.iota(jnp.int32, 16)
                q = (lanes >> tsh) << tsh
                e0 = 2 * (q + j16)
                e1 = 2 * (q + i16) + 1
                e2 = 2 * (q + jnp.maximum(i16 - 1, 0))
                e3 = 2 * (q + jnp.minimum(j16 + 1, T - 1)) + 1
                idx[0 * n_ch + h, pl.ds(p * 16, 16)] = e0
                idx[1 * n_ch + h, pl.ds(p * 16, 16)] = e1
                idx[2 * n_ch + h, pl.ds(p * 16, 16)] = e2
                idx[3 * n_ch + h, pl.ds(p * 16, 16)] = e3
                return _

            lax.fori_loop(0, C // 16, comp, None)

        order = [(h, blk) for h in range(n_ch) for blk in range(4)]
        bufs = (buf0, buf1)
        sems = (sem0, sem1)
        prev = None
        for t, (h, blk) in enumerate(order):
            p = t % 2
            cp = pltpu.async_copy(
                table_h.at[idx.at[blk * n_ch + h]], bufs[p], sems[p])
            if prev is not None:
                pt, pcp = prev
                pcp.wait()
                ph, pblk = order[pt]
                pltpu.sync_copy(
                    bufs[pt % 2],
                    a_h.at[pl.ds(base + ph * C, C),
                           pl.ds(pblk * HID, HID)])
            prev = (t, cp)
        pt, pcp = prev
        pcp.wait()
        ph, pblk = order[pt]
        pltpu.sync_copy(
            bufs[pt % 2],
            a_h.at[pl.ds(base + ph * C, C), pl.ds(pblk * HID, HID)])

    return k(table, i_arr, j_arr)


def _matmul_half(A, masks, Wr, b2, S, OUT, m_off, prev_out=None):
    BM = 1024
    K = 4 * HID
    SH = A.shape[0]

    def mm(*refs):
        if prev_out is None:
            a_ref, m_ref, w_ref, b_ref, o_ref = refs
        else:
            a_ref, m_ref, w_ref, b_ref, _prev, o_ref = refs
        acc = b_ref[...]
        for c in range(4):
            blk = a_ref[:, c * HID:(c + 1) * HID] * m_ref[:, c][:, None]
            acc = acc + lax.dot_general(
                blk.astype(jnp.bfloat16),
                w_ref[:, c * HID:(c + 1) * HID],
                (((1,), (1,)), ((), ())),
                preferred_element_type=jnp.float32)
        o_ref[...] = acc

    in_specs = [
        pl.BlockSpec((BM, K), lambda m: (m, 0)),
        pl.BlockSpec((BM, 4), lambda m: (m, 0)),
        pl.BlockSpec((OUT, K), lambda m: (0, 0)),
        pl.BlockSpec((1, OUT), lambda m: (0, 0)),
    ]
    operands = [A, masks, Wr, b2]
    kwargs = {}
    if prev_out is not None:
        in_specs.append(pl.BlockSpec(memory_space=pltpu.MemorySpace.HBM))
        operands.append(prev_out)
        kwargs["input_output_aliases"] = {4: 0}

    moff_tiles = m_off // BM
    return pl.pallas_call(
        mm,
        grid=(SH // BM,),
        in_specs=in_specs,
        out_specs=pl.BlockSpec((BM, OUT), lambda m: (m + moff_tiles, 0)),
        out_shape=jax.ShapeDtypeStruct((S, OUT), jnp.float32),
        compiler_params=pltpu.CompilerParams(
            dimension_semantics=("parallel",)),
        **kwargs,
    )(*operands)


def kernel(input, span_idxs, W, b):
    B, T, two_h = input.shape
    OUT = W.shape[0]
    S = B * T
    SH = S // 2

    table = input.reshape(2 * S, HID)

    si = span_idxs.astype(jnp.int32)
    i_arr = si[..., 0].reshape(S)
    j_arr = si[..., 1].reshape(S)

    skip = (i_arr == 0) & (j_arr == 0)
    m01 = ~skip
    m2 = i_arr > 0
    m3 = m01 & (j_arr < T - 1)
    masks = jnp.stack([m01, m01, m2, m3], axis=1).astype(jnp.float32)

    W1 = W[:, 0:HID]
    W2 = W[:, HID:2 * HID]
    W3 = W[:, 2 * HID:3 * HID]
    W4 = W[:, 3 * HID:4 * HID]
    Wr = jnp.concatenate([W1, W2, W3 - W1, W4 - W2],
                         axis=1).astype(jnp.bfloat16)
    b2 = b.reshape(1, OUT)

    A1 = _gather_half(table, i_arr[:SH], j_arr[:SH], 0, SH, S, T)
    A2 = _gather_half(table, i_arr[SH:], j_arr[SH:], SH, SH, S, T)

    out1 = _matmul_half(A1, masks[:SH], Wr, b2, S, OUT, 0)
    out = _matmul_half(A2, masks[SH:], Wr, b2, S, OUT, SH, prev_out=out1)
    return out.reshape(B, T, OUT)


# speedup vs baseline: 1.0780x; 1.0003x over previous
import functools

import jax
import jax.numpy as jnp
from jax import lax
from jax.experimental import pallas as pl
from jax.experimental.pallas import tpu as pltpu
from jax.experimental.pallas import tpu_sc as plsc

HID = 512


def _gather_half(table, i_arr, j_arr, row0, SH, S, T):
    info = plsc.get_sparse_core_info()
    NC, NS = info.num_cores, info.num_subcores
    NW = NC * NS
    per_w = SH // NW
    C = 64
    n_ch = per_w // C
    tsh = (T - 1).bit_length()
    assert T == (1 << tsh)
    mesh = plsc.VectorSubcoreMesh(core_axis_name="c", subcore_axis_name="s")

    @functools.partial(
        pl.kernel,
        mesh=mesh,
        out_type=jax.ShapeDtypeStruct((SH, 4 * HID), jnp.float32),
        scratch_types=[
            pltpu.VMEM((per_w,), jnp.int32),
            pltpu.VMEM((per_w,), jnp.int32),
            pltpu.VMEM((4 * n_ch, C), jnp.int32),
            pltpu.VMEM((C, HID), jnp.float32),
            pltpu.VMEM((C, HID), jnp.float32),
            pltpu.SemaphoreType.DMA,
            pltpu.SemaphoreType.DMA,
        ],
    )
    def k(table_h, i_h, j_h, a_h, iv, jv, idx, buf0, buf1, sem0, sem1):
        wid = lax.axis_index("s") * NC + lax.axis_index("c")
        base = wid * per_w
        pltpu.sync_copy(i_h.at[pl.ds(base, per_w)], iv)
        pltpu.sync_copy(j_h.at[pl.ds(base, per_w)], jv)

        for h in range(n_ch):
            def comp(p, _, h=h):
                off = h * C + p * 16
                i16 = iv[pl.ds(off, 16)]
                j16 = jv[pl.ds(off, 16)]
                lanes = row0 + base + off + lax.iota(jnp.int32, 16)
                q = (lanes >> tsh) << tsh
                e0 = 2 * (q + j16)
                e1 = 2 * (q + i16) + 1
                e2 = 2 * (q + jnp.maximum(i16 - 1, 0))
                e3 = 2 * (q + jnp.minimum(j16 + 1, T - 1)) + 1
                idx[0 * n_ch + h, pl.ds(p * 16, 16)] = e0
                idx[1 * n_ch + h, pl.ds(p * 16, 16)] = e1
                idx[2 * n_ch + h, pl.ds(p * 16, 16)] = e2
                idx[3 * n_ch + h, pl.ds(p * 16, 16)] = e3
                return _

            lax.fori_loop(0, C // 16, comp, None)

        order = [(h, blk) for h in range(n_ch) for blk in range(4)]
        bufs = (buf0, buf1)
        sems = (sem0, sem1)
        prev = None
        for t, (h, blk) in enumerate(order):
            p = t % 2
            cp = pltpu.async_copy(
                table_h.at[idx.at[blk * n_ch + h]], bufs[p], sems[p])
            if prev is not None:
                pt, pcp = prev
                pcp.wait()
                ph, pblk = order[pt]
                pltpu.sync_copy(
                    bufs[pt % 2],
                    a_h.at[pl.ds(base + ph * C, C),
                           pl.ds(pblk * HID, HID)])
            prev = (t, cp)
        pt, pcp = prev
        pcp.wait()
        ph, pblk = order[pt]
        pltpu.sync_copy(
            bufs[pt % 2],
            a_h.at[pl.ds(base + ph * C, C), pl.ds(pblk * HID, HID)])

    return k(table, i_arr, j_arr)


def _matmul_half(A, masks, Wr, b2, S, OUT, m_off, prev_out=None):
    BM = 512
    K = 4 * HID
    SH = A.shape[0]

    def mm(*refs):
        if prev_out is None:
            a_ref, m_ref, w_ref, b_ref, o_ref = refs
        else:
            a_ref, m_ref, w_ref, b_ref, _prev, o_ref = refs
        parts = []
        for c in range(4):
            blk = a_ref[:, c * HID:(c + 1) * HID] * m_ref[:, c][:, None]
            parts.append(blk.astype(jnp.bfloat16))
        a = jnp.concatenate(parts, axis=1)
        o_ref[...] = lax.dot_general(
            a, w_ref[...],
            (((1,), (1,)), ((), ())),
            preferred_element_type=jnp.float32) + b_ref[...]

    in_specs = [
        pl.BlockSpec((BM, K), lambda m: (m, 0)),
        pl.BlockSpec((BM, 4), lambda m: (m, 0)),
        pl.BlockSpec((OUT, K), lambda m: (0, 0)),
        pl.BlockSpec((1, OUT), lambda m: (0, 0)),
    ]
    operands = [A, masks, Wr, b2]
    kwargs = {}
    if prev_out is not None:
        in_specs.append(pl.BlockSpec(memory_space=pltpu.MemorySpace.HBM))
        operands.append(prev_out)
        kwargs["input_output_aliases"] = {4: 0}

    moff_tiles = m_off // BM
    return pl.pallas_call(
        mm,
        grid=(SH // BM,),
        in_specs=in_specs,
        out_specs=pl.BlockSpec((BM, OUT), lambda m: (m + moff_tiles, 0)),
        out_shape=jax.ShapeDtypeStruct((S, OUT), jnp.float32),
        compiler_params=pltpu.CompilerParams(
            dimension_semantics=("parallel",)),
        **kwargs,
    )(*operands)


def kernel(input, span_idxs, W, b):
    B, T, two_h = input.shape
    OUT = W.shape[0]
    S = B * T
    SH = S // 2

    table = input.reshape(2 * S, HID)

    si = span_idxs.astype(jnp.int32)
    i_arr = si[..., 0].reshape(S)
    j_arr = si[..., 1].reshape(S)

    skip = (i_arr == 0) & (j_arr == 0)
    m01 = ~skip
    m2 = i_arr > 0
    m3 = m01 & (j_arr < T - 1)
    masks = jnp.stack([m01, m01, m2, m3], axis=1).astype(jnp.float32)

    W1 = W[:, 0:HID]
    W2 = W[:, HID:2 * HID]
    W3 = W[:, 2 * HID:3 * HID]
    W4 = W[:, 3 * HID:4 * HID]
    Wr = jnp.concatenate([W1, W2, W3 - W1, W4 - W2],
                         axis=1).astype(jnp.bfloat16)
    b2 = b.reshape(1, OUT)

    A1 = _gather_half(table, i_arr[:SH], j_arr[:SH], 0, SH, S, T)
    A2 = _gather_half(table, i_arr[SH:], j_arr[SH:], SH, SH, S, T)

    out1 = _matmul_half(A1, masks[:SH], Wr, b2, S, OUT, 0)
    out = _matmul_half(A2, masks[SH:], Wr, b2, S, OUT, SH, prev_out=out1)
    return out.reshape(B, T, OUT)
